# baseline (speedup 1.0000x reference)
import functools

import jax
import jax.numpy as jnp
from jax import lax
from jax.experimental import pallas as pl
from jax.experimental.pallas import tpu as pltpu
from jax.experimental.pallas import tpu_sc as plsc

_VOCAB = 100000
_EMBED = 128
_BATCH = 4096
_SEQ = 200

_NC = 2
_NS = 16
_NW = _NC * _NS
_ROWS_PER_W = _BATCH // _NW
_GROUPS = _ROWS_PER_W // 16
_TBLK = 8
_VPAD = 100352
_VBLK = 7168


def _matvec_body(t_ref, w_ref, b_ref, o_ref, o2_ref):
    t3 = t_ref[...].reshape(_VBLK // _EMBED, _EMBED, _EMBED)
    r = jax.lax.dot_general(
        w_ref[...], t3, (((0,), (2,)), ((), ())),
        preferred_element_type=jnp.float32,
    )
    o_ref[...] = r * (1.0 / _SEQ)
    o2_ref[...] = jnp.full((16,), b_ref[0], jnp.float32)


def _matvec(table, fc_w, fc_b):
    return pl.pallas_call(
        _matvec_body,
        grid=(_VPAD // _VBLK,),
        in_specs=[
            pl.BlockSpec((_VBLK, _EMBED), lambda i: (i, 0)),
            pl.BlockSpec((_EMBED,), lambda i: (0,)),
            pl.BlockSpec((1,), lambda i: (0,)),
        ],
        out_specs=[
            pl.BlockSpec((_VBLK // _EMBED, _EMBED), lambda i: (i, 0)),
            pl.BlockSpec((16,), lambda i: (0,)),
        ],
        out_shape=[
            jax.ShapeDtypeStruct((_VPAD // _EMBED, _EMBED), jnp.float32),
            jax.ShapeDtypeStruct((16,), jnp.float32),
        ],
    )(table, fc_w, fc_b)


def _sc_pool_body(xt_hbm, v_hbm, b_hbm, out_hbm, x_l, v_l, b_l, o_l, sem):
    cid = lax.axis_index("c")
    sid = lax.axis_index("s")
    w = sid * _NC + cid
    base = w * _ROWS_PER_W

    c1 = pltpu.async_copy(xt_hbm.at[:, pl.ds(base, _ROWS_PER_W)], x_l, sem)
    c2 = pltpu.async_copy(v_hbm.at[pl.ds(0, _VOCAB)], v_l, sem)
    c3 = pltpu.async_copy(b_hbm, b_l, sem)
    c1.wait()
    c2.wait()
    c3.wait()

    bias = b_l[...]
    init = tuple(bias for _ in range(_GROUPS))

    @plsc.parallel_loop(0, _SEQ // _TBLK, 1, carry=init)
    def accs(i, accs):
        accs = list(accs)
        t0 = pl.multiple_of(i * _TBLK, _TBLK)
        for k in range(_TBLK):
            for g in range(_GROUPS):
                idx = x_l[t0 + k, pl.ds(g * 16, 16)]
                vals = plsc.load_gather(v_l, [idx])
                accs[g] = accs[g] + vals
        return tuple(accs)
    for g in range(_GROUPS):
        o_l[pl.ds(g * 16, 16)] = accs[g]
    pltpu.sync_copy(o_l, out_hbm.at[pl.ds(base, _ROWS_PER_W)])


_sc_pool = functools.partial(
    pl.kernel,
    mesh=plsc.VectorSubcoreMesh(core_axis_name="c", subcore_axis_name="s"),
    out_type=jax.ShapeDtypeStruct((_BATCH,), jnp.float32),
    compiler_params=pltpu.CompilerParams(needs_layout_passes=False),
    scratch_types=[
        pltpu.VMEM((_SEQ, _ROWS_PER_W), jnp.int32),
        pltpu.VMEM((_VOCAB,), jnp.float32),
        pltpu.VMEM((16,), jnp.float32),
        pltpu.VMEM((_ROWS_PER_W,), jnp.float32),
        pltpu.SemaphoreType.DMA,
    ],
)(_sc_pool_body)


def kernel(x, table, fc_w, fc_b):
    xt = jnp.swapaxes(x.astype(jnp.int32), 0, 1)
    v2d, b16 = _matvec(table, fc_w.reshape(_EMBED), fc_b)
    out = _sc_pool(xt, v2d.reshape(_VPAD), b16)
    return out.reshape(_BATCH, 1)


# speedup vs baseline: 1.0710x; 1.0710x over previous
import functools

import jax
import jax.numpy as jnp
from jax import lax
from jax.experimental import pallas as pl
from jax.experimental.pallas import tpu as pltpu
from jax.experimental.pallas import tpu_sc as plsc

_VOCAB = 100000
_EMBED = 128
_BATCH = 4096
_SEQ = 200

_NC = 2
_NS = 16
_NW = _NC * _NS
_ROWS_PER_W = _BATCH // _NW
_GROUPS = _ROWS_PER_W // 16
_TBLK = 4
_VPAD = 100352
_VBLK = 14336


def _matvec_body(t_ref, w_ref, b_ref, o_ref, o2_ref):
    t3 = t_ref[...].reshape(_VBLK // _EMBED, _EMBED, _EMBED)
    r = jax.lax.dot_general(
        w_ref[...], t3, (((0,), (2,)), ((), ())),
        preferred_element_type=jnp.float32,
    )
    o_ref[...] = r * (1.0 / _SEQ)
    o2_ref[...] = jnp.full((16,), b_ref[0], jnp.float32)


def _matvec(table, fc_w, fc_b):
    return pl.pallas_call(
        _matvec_body,
        grid=(_VPAD // _VBLK,),
        in_specs=[
            pl.BlockSpec((_VBLK, _EMBED), lambda i: (i, 0)),
            pl.BlockSpec((_EMBED,), lambda i: (0,)),
            pl.BlockSpec((1,), lambda i: (0,)),
        ],
        out_specs=[
            pl.BlockSpec((_VBLK // _EMBED, _EMBED), lambda i: (i, 0)),
            pl.BlockSpec((16,), lambda i: (0,)),
        ],
        out_shape=[
            jax.ShapeDtypeStruct((_VPAD // _EMBED, _EMBED), jnp.float32),
            jax.ShapeDtypeStruct((16,), jnp.float32),
        ],
    )(table, fc_w, fc_b)


def _sc_pool_body(xt_hbm, v_hbm, b_hbm, out_hbm, x_l, v_l, b_l, o_l, sem):
    cid = lax.axis_index("c")
    sid = lax.axis_index("s")
    w = sid * _NC + cid
    base = w * _ROWS_PER_W

    c1 = pltpu.async_copy(xt_hbm.at[:, pl.ds(base, _ROWS_PER_W)], x_l, sem)
    c2 = pltpu.async_copy(v_hbm.at[pl.ds(0, _VOCAB)], v_l, sem)
    c3 = pltpu.async_copy(b_hbm, b_l, sem)
    c1.wait()
    c2.wait()
    c3.wait()

    bias = b_l[...]
    init = tuple(bias for _ in range(_GROUPS))

    def step(i, accs):
        accs = list(accs)
        t0 = i * _TBLK
        for k in range(_TBLK):
            for g in range(_GROUPS):
                idx = x_l[t0 + k, pl.ds(g * 16, 16)]
                vals = plsc.load_gather(v_l, [idx])
                accs[g] = accs[g] + vals
        return tuple(accs)

    accs = lax.fori_loop(0, _SEQ // _TBLK, step, init)
    for g in range(_GROUPS):
        o_l[pl.ds(g * 16, 16)] = accs[g]
    pltpu.sync_copy(o_l, out_hbm.at[pl.ds(base, _ROWS_PER_W)])


_sc_pool = functools.partial(
    pl.kernel,
    mesh=plsc.VectorSubcoreMesh(core_axis_name="c", subcore_axis_name="s"),
    out_type=jax.ShapeDtypeStruct((_BATCH,), jnp.float32),
    compiler_params=pltpu.CompilerParams(needs_layout_passes=False),
    scratch_types=[
        pltpu.VMEM((_SEQ, _ROWS_PER_W), jnp.int32),
        pltpu.VMEM((_VOCAB,), jnp.float32),
        pltpu.VMEM((16,), jnp.float32),
        pltpu.VMEM((_ROWS_PER_W,), jnp.float32),
        pltpu.SemaphoreType.DMA,
    ],
)(_sc_pool_body)


def kernel(x, table, fc_w, fc_b):
    xt = jnp.swapaxes(x.astype(jnp.int32), 0, 1)
    v2d, b16 = _matvec(table, fc_w.reshape(_EMBED), fc_b)
    out = _sc_pool(xt, v2d.reshape(_VPAD), b16)
    return out.reshape(_BATCH, 1)


# speedup vs baseline: 1.0722x; 1.0012x over previous
import functools

import jax
import jax.numpy as jnp
from jax import lax
from jax.experimental import pallas as pl
from jax.experimental.pallas import tpu as pltpu
from jax.experimental.pallas import tpu_sc as plsc

_VOCAB = 100000
_EMBED = 128
_BATCH = 4096
_SEQ = 200

_NC = 2
_NS = 16
_NW = _NC * _NS
_ROWS_PER_W = _BATCH // _NW
_GROUPS = _ROWS_PER_W // 16
_TBLK = 4
_VPAD = 100352
_VBLK = 14336


def _matvec_body(t_ref, w_ref, b_ref, o_ref, o2_ref):
    t3 = t_ref[...].reshape(_VBLK // _EMBED, _EMBED, _EMBED)
    r = jax.lax.dot_general(
        w_ref[...], t3, (((0,), (2,)), ((), ())),
        preferred_element_type=jnp.float32,
    )
    o_ref[...] = r * (1.0 / _SEQ)
    o2_ref[...] = jnp.full((16,), b_ref[0], jnp.float32)


def _matvec(table, fc_w, fc_b):
    return pl.pallas_call(
        _matvec_body,
        grid=(_VPAD // _VBLK,),
        in_specs=[
            pl.BlockSpec((_VBLK, _EMBED), lambda i: (i, 0)),
            pl.BlockSpec((_EMBED,), lambda i: (0,)),
            pl.BlockSpec((1,), lambda i: (0,)),
        ],
        out_specs=[
            pl.BlockSpec((_VBLK // _EMBED, _EMBED), lambda i: (i, 0)),
            pl.BlockSpec((16,), lambda i: (0,)),
        ],
        out_shape=[
            jax.ShapeDtypeStruct((_VPAD // _EMBED, _EMBED), jnp.float32),
            jax.ShapeDtypeStruct((16,), jnp.float32),
        ],
    )(table, fc_w, fc_b)


def _sc_pool_body(xt_hbm, v_hbm, b_hbm, out_hbm, x_l, v_l, b_l, o_l, sem):
    cid = lax.axis_index("c")
    sid = lax.axis_index("s")
    w = sid * _NC + cid
    base = w * _ROWS_PER_W

    c1 = pltpu.async_copy(xt_hbm.at[:, pl.ds(base, _ROWS_PER_W)], x_l, sem)
    c2 = pltpu.async_copy(v_hbm.at[pl.ds(0, _VOCAB)], v_l, sem)
    c3 = pltpu.async_copy(b_hbm, b_l, sem)
    c1.wait()
    c2.wait()
    c3.wait()

    bias = b_l[...]
    init = tuple(bias for _ in range(_GROUPS))

    def step(i, accs):
        accs = list(accs)
        t0 = i * _TBLK
        for k in range(_TBLK):
            for g in range(_GROUPS):
                idx = x_l[t0 + k, pl.ds(g * 16, 16)]
                vals = plsc.load_gather(v_l, [idx])
                accs[g] = accs[g] + vals
        return tuple(accs)

    accs = lax.fori_loop(0, _SEQ // _TBLK, step, init)
    for g in range(_GROUPS):
        o_l[pl.ds(g * 16, 16)] = accs[g]
    pltpu.sync_copy(o_l, out_hbm.at[pl.ds(base, _ROWS_PER_W)])


_sc_pool = functools.partial(
    pl.kernel,
    mesh=plsc.VectorSubcoreMesh(core_axis_name="c", subcore_axis_name="s"),
    out_type=jax.ShapeDtypeStruct((_BATCH,), jnp.float32),
    compiler_params=pltpu.CompilerParams(
        needs_layout_passes=False, skip_device_barrier=True
    ),
    scratch_types=[
        pltpu.VMEM((_SEQ, _ROWS_PER_W), jnp.int32),
        pltpu.VMEM((_VOCAB,), jnp.float32),
        pltpu.VMEM((16,), jnp.float32),
        pltpu.VMEM((_ROWS_PER_W,), jnp.float32),
        pltpu.SemaphoreType.DMA,
    ],
)(_sc_pool_body)


def kernel(x, table, fc_w, fc_b):
    xt = jnp.swapaxes(x.astype(jnp.int32), 0, 1)
    v2d, b16 = _matvec(table, fc_w.reshape(_EMBED), fc_b)
    out = _sc_pool(xt, v2d.reshape(_VPAD), b16)
    return out.reshape(_BATCH, 1)


# speedup vs baseline: 1.0770x; 1.0044x over previous
import functools

import jax
import jax.numpy as jnp
from jax import lax
from jax.experimental import pallas as pl
from jax.experimental.pallas import tpu as pltpu
from jax.experimental.pallas import tpu_sc as plsc

_VOCAB = 100000
_EMBED = 128
_BATCH = 4096
_SEQ = 200

_NC = 2
_NS = 16
_NW = _NC * _NS
_ROWS_PER_W = _BATCH // _NW
_GROUPS = _ROWS_PER_W // 16
_TBLK = 4
_VPAD = 100352
_VBLK = 14336


def _matvec_body(t_ref, w_ref, b_ref, o_ref, o2_ref):
    t3 = t_ref[...].reshape(_VBLK // _EMBED, _EMBED, _EMBED)
    r = jax.lax.dot_general(
        w_ref[...], t3, (((0,), (2,)), ((), ())),
        preferred_element_type=jnp.float32,
    )
    o_ref[...] = r * (1.0 / _SEQ)
    o2_ref[...] = jnp.full((16,), b_ref[0], jnp.float32)


def _matvec(table, fc_w, fc_b):
    return pl.pallas_call(
        _matvec_body,
        grid=(_VPAD // _VBLK,),
        in_specs=[
            pl.BlockSpec((_VBLK, _EMBED), lambda i: (i, 0)),
            pl.BlockSpec((_EMBED,), lambda i: (0,)),
            pl.BlockSpec((1,), lambda i: (0,)),
        ],
        out_specs=[
            pl.BlockSpec((_VBLK // _EMBED, _EMBED), lambda i: (i, 0)),
            pl.BlockSpec((16,), lambda i: (0,)),
        ],
        out_shape=[
            jax.ShapeDtypeStruct((_VPAD // _EMBED, _EMBED), jnp.float32),
            jax.ShapeDtypeStruct((16,), jnp.float32),
        ],
    )(table, fc_w, fc_b)


def _sc_pool_body(xt_hbm, v_hbm, b_hbm, out_hbm, x_l, v_l, b_l, o_l, sem):
    cid = lax.axis_index("c")
    sid = lax.axis_index("s")
    w = sid * _NC + cid
    base = w * _ROWS_PER_W

    c1 = pltpu.async_copy(xt_hbm.at[:, pl.ds(base, _ROWS_PER_W)], x_l, sem)
    c2 = pltpu.async_copy(v_hbm.at[pl.ds(0, _VOCAB)], v_l, sem)
    c3 = pltpu.async_copy(b_hbm, b_l, sem)
    c1.wait()
    c2.wait()
    c3.wait()

    bias = b_l[...]
    init = tuple(bias for _ in range(_GROUPS))

    def step(i, accs):
        accs = list(accs)
        t0 = i * _TBLK
        for k in range(_TBLK):
            for g in range(_GROUPS):
                idx = x_l[t0 + k, pl.ds(g * 16, 16)]
                vals = plsc.load_gather(v_l, [idx])
                accs[g] = accs[g] + vals
        return tuple(accs)

    accs = lax.fori_loop(0, _SEQ // _TBLK, step, init)
    for g in range(_GROUPS):
        o_l[pl.ds(g * 16, 16)] = accs[g]
    pltpu.sync_copy(o_l, out_hbm.at[pl.ds(base, _ROWS_PER_W)])


_sc_pool = functools.partial(
    pl.kernel,
    mesh=plsc.VectorSubcoreMesh(core_axis_name="c", subcore_axis_name="s"),
    out_type=jax.ShapeDtypeStruct((_BATCH,), jnp.float32),
    compiler_params=pltpu.CompilerParams(needs_layout_passes=False),
    scratch_types=[
        pltpu.VMEM((_SEQ, _ROWS_PER_W), jnp.int32),
        pltpu.VMEM((_VOCAB,), jnp.float32),
        pltpu.VMEM((16,), jnp.float32),
        pltpu.VMEM((_ROWS_PER_W,), jnp.float32),
        pltpu.SemaphoreType.DMA,
    ],
)(_sc_pool_body)


def kernel(x, table, fc_w, fc_b):
    xt = jnp.swapaxes(x.astype(jnp.int32), 0, 1)
    v2d, b16 = _matvec(table, fc_w.reshape(_EMBED), fc_b)
    out = _sc_pool(xt, v2d.reshape(_VPAD), b16)
    return out.reshape(_BATCH, 1)
